# baseline (speedup 1.0000x reference)
import functools

import jax
import jax.numpy as jnp
import numpy as np
from jax.experimental import pallas as pl
from jax.experimental.pallas import tpu as pltpu

_N = 20000
_C = 81
_IMW = 1024.0
_CLIP = float(np.log(1000.0 / 16.0))
_R2D = float(180.0 / np.pi)

_DN_T = (((0,), (0,)), ((), ()))
_DN_N = (((1,), (0,)), ((), ()))

_ROWMAJOR = (0, 1, 5)


def _split(x):
    hi = x.astype(jnp.bfloat16).astype(jnp.float32)
    return jnp.concatenate([hi, x - hi], axis=0)


def _eye(m):
    return (jax.lax.broadcasted_iota(jnp.int32, (m, m), 0) ==
            jax.lax.broadcasted_iota(jnp.int32, (m, m), 1)).astype(jnp.float32)


def _sel(j):
    row = jax.lax.broadcasted_iota(jnp.int32, (_C * 5, _C), 0)
    col = jax.lax.broadcasted_iota(jnp.int32, (_C * 5, _C), 1)
    return (row == 5 * col + j).astype(jnp.float32)


def _tdot(x, sel):
    sel2 = jnp.concatenate([sel, sel], axis=0)
    return jax.lax.dot_general(
        _split(x), sel2, _DN_T, preferred_element_type=jnp.float32)


def _ndot(sel, x):
    sel2 = jnp.concatenate([sel.T, sel.T], axis=1)
    return jax.lax.dot_general(
        sel2, _split(x), _DN_N, preferred_element_type=jnp.float32)


def _body(logits_ref, codes_ref, props_ref, *out_refs):
    logits = logits_ref[...]
    m = jnp.max(logits, axis=0, keepdims=True)
    p = jnp.exp(logits - m)
    s = jnp.sum(p, axis=0, keepdims=True)
    prob = p / s
    if 5 in _ROWMAJOR:
        out_refs[5][...] = _tdot(prob, _eye(_C))
    else:
        out_refs[5][...] = prob

    codes = codes_ref[...]
    props = props_ref[...]

    cxr = props[0:1, :]
    cyr = props[1:2, :]
    wr = props[2:3, :]
    hr = props[3:4, :]
    ar = props[4:5, :]
    props_t = _tdot(props, _eye(5))
    stats_rm = (props_t[:, 0:1], props_t[:, 1:2], props_t[:, 2:3],
                props_t[:, 3:4], props_t[:, 4:5])
    stats_cm = (cxr, cyr, wr, hr, ar)

    for j in range(5):
        if j in _ROWMAJOR:
            d = _tdot(codes, _sel(j))
            cx, cy, w, h, a = stats_rm
        else:
            d = _ndot(_sel(j), codes)
            cx, cy, w, h, a = stats_cm
        if j == 0:
            out = jnp.clip(d * 0.1 * w + cx, 0.0, _IMW - 1.0)
        elif j == 1:
            out = jnp.clip(d * 0.1 * h + cy, 0.0, _IMW - 1.0)
        elif j == 2:
            out = jnp.exp(jnp.minimum(d * 0.2, _CLIP)) * w
        elif j == 3:
            out = jnp.exp(jnp.minimum(d * 0.2, _CLIP)) * h
        else:
            out = d * _R2D + a
        out_refs[j][...] = out


@functools.partial(jax.jit, static_argnums=(3,))
def _run(class_logits, box_regression, proposals, block_cols):
    n = class_logits.shape[0]
    lg_t = jnp.transpose(class_logits)
    codes_t = jnp.transpose(box_regression)
    props_t = jnp.transpose(proposals)
    grid = (pl.cdiv(n, block_cols),)
    rm_spec = pl.BlockSpec((block_cols, _C), lambda i: (i, 0))
    rm_shape = jax.ShapeDtypeStruct((n, _C), jnp.float32)
    cm_spec = pl.BlockSpec((_C, block_cols), lambda i: (0, i))
    cm_shape = jax.ShapeDtypeStruct((_C, n), jnp.float32)
    out_specs = [rm_spec if j in _ROWMAJOR else cm_spec for j in range(6)]
    out_shape = [rm_shape if j in _ROWMAJOR else cm_shape for j in range(6)]
    outs = pl.pallas_call(
        _body,
        grid=grid,
        in_specs=[
            pl.BlockSpec((_C, block_cols), lambda i: (0, i)),
            pl.BlockSpec((_C * 5, block_cols), lambda i: (0, i)),
            pl.BlockSpec((5, block_cols), lambda i: (0, i)),
        ],
        out_specs=out_specs,
        out_shape=out_shape,
        compiler_params=pltpu.CompilerParams(
            dimension_semantics=("parallel",),
        ),
    )(lg_t, codes_t, props_t)
    planes = [outs[j] if j in _ROWMAJOR else jnp.transpose(outs[j])
              for j in range(6)]
    pred = jnp.stack(planes[:5], axis=2)
    boxes = pred.reshape(-1, 5)
    scores = planes[5].reshape(-1)
    return boxes, scores


def kernel(class_logits, box_regression, proposals, num_of_fwd_left=0):
    return _run(class_logits, box_regression, proposals, 2048)


# speedup vs baseline: 1.0356x; 1.0356x over previous
import functools

import jax
import jax.numpy as jnp
import numpy as np
from jax.experimental import pallas as pl
from jax.experimental.pallas import tpu as pltpu

_N = 20000
_C = 81
_IMW = 1024.0
_CLIP = float(np.log(1000.0 / 16.0))
_R2D = float(180.0 / np.pi)

_DN_T = (((0,), (0,)), ((), ()))


def _split(x):
    hi = x.astype(jnp.bfloat16).astype(jnp.float32)
    return jnp.concatenate([hi, x - hi], axis=0)


def _transposing_dot(x, sel):
    sel2 = jnp.concatenate([sel, sel], axis=0)
    return jax.lax.dot_general(
        _split(x), sel2, _DN_T, preferred_element_type=jnp.float32)


def _eye(m):
    return (jax.lax.broadcasted_iota(jnp.int32, (m, m), 0) ==
            jax.lax.broadcasted_iota(jnp.int32, (m, m), 1)).astype(jnp.float32)


def _body(logits_ref, codes_ref, props_ref, px_ref, py_ref, pw_ref, ph_ref,
          pa_ref, scores_ref):
    logits = logits_ref[...]
    m = jnp.max(logits, axis=0, keepdims=True)
    p = jnp.exp(logits - m)
    s = jnp.sum(p, axis=0, keepdims=True)
    prob = p / s
    scores_ref[...] = _transposing_dot(prob, _eye(_C))

    codes = codes_ref[...]
    props = props_ref[...]

    row = jax.lax.broadcasted_iota(jnp.int32, (_C * 5, _C), 0)
    col = jax.lax.broadcasted_iota(jnp.int32, (_C * 5, _C), 1)

    def plane(j):
        return _transposing_dot(codes, (row == 5 * col + j).astype(jnp.float32))

    props_t = _transposing_dot(props, _eye(5))
    cx = props_t[:, 0:1]
    cy = props_t[:, 1:2]
    w = props_t[:, 2:3]
    h = props_t[:, 3:4]
    a = props_t[:, 4:5]

    px_ref[...] = jnp.clip(plane(0) * 0.1 * w + cx, 0.0, _IMW - 1.0)
    py_ref[...] = jnp.clip(plane(1) * 0.1 * h + cy, 0.0, _IMW - 1.0)
    pw_ref[...] = jnp.exp(jnp.minimum(plane(2) * 0.2, _CLIP)) * w
    ph_ref[...] = jnp.exp(jnp.minimum(plane(3) * 0.2, _CLIP)) * h
    pa_ref[...] = plane(4) * _R2D + a


@functools.partial(jax.jit, static_argnums=(3,))
def _run(class_logits, box_regression, proposals, block_cols):
    n = class_logits.shape[0]
    lg_t = jnp.transpose(class_logits)
    codes_t = jnp.transpose(box_regression)
    props_t = jnp.transpose(proposals)
    grid = (pl.cdiv(n, block_cols),)
    out_spec = pl.BlockSpec((block_cols, _C), lambda i: (i, 0))
    out_shape = jax.ShapeDtypeStruct((n, _C), jnp.float32)
    px, py, pw, ph, pa, scores_rm = pl.pallas_call(
        _body,
        grid=grid,
        in_specs=[
            pl.BlockSpec((_C, block_cols), lambda i: (0, i)),
            pl.BlockSpec((_C * 5, block_cols), lambda i: (0, i)),
            pl.BlockSpec((5, block_cols), lambda i: (0, i)),
        ],
        out_specs=[out_spec] * 6,
        out_shape=[out_shape] * 6,
        compiler_params=pltpu.CompilerParams(
            dimension_semantics=("parallel",),
        ),
    )(lg_t, codes_t, props_t)
    pred = jnp.stack([px, py, pw, ph, pa], axis=2)
    boxes = pred.reshape(-1, 5)
    scores = scores_rm.reshape(-1)
    return boxes, scores


def kernel(class_logits, box_regression, proposals, num_of_fwd_left=0):
    return _run(class_logits, box_regression, proposals, 1024)


# speedup vs baseline: 1.1150x; 1.0767x over previous
import functools

import jax
import jax.numpy as jnp
import numpy as np
from jax.experimental import pallas as pl
from jax.experimental.pallas import tpu as pltpu

_N = 20000
_C = 81
_IMW = 1024.0
_CLIP = float(np.log(1000.0 / 16.0))
_R2D = float(180.0 / np.pi)

_DN_T = (((0,), (0,)), ((), ()))


def _transposing_dot(x, sel):
    return jax.lax.dot_general(
        x, sel, _DN_T, preferred_element_type=jnp.float32)


def _eye(m):
    return (jax.lax.broadcasted_iota(jnp.int32, (m, m), 0) ==
            jax.lax.broadcasted_iota(jnp.int32, (m, m), 1)).astype(jnp.float32)


def _body(logits_ref, codes_ref, props_ref, px_ref, py_ref, pw_ref, ph_ref,
          pa_ref, scores_ref):
    logits = logits_ref[...]
    m = jnp.max(logits, axis=0, keepdims=True)
    p = jnp.exp(logits - m)
    s = jnp.sum(p, axis=0, keepdims=True)
    prob = p / s
    scores_ref[...] = _transposing_dot(prob, _eye(_C))

    codes = codes_ref[...]
    props = props_ref[...]

    row = jax.lax.broadcasted_iota(jnp.int32, (_C * 5, _C), 0)
    col = jax.lax.broadcasted_iota(jnp.int32, (_C * 5, _C), 1)

    def plane(j):
        return _transposing_dot(codes, (row == 5 * col + j).astype(jnp.float32))

    props_t = _transposing_dot(props, _eye(5))
    cx = props_t[:, 0:1]
    cy = props_t[:, 1:2]
    w = props_t[:, 2:3]
    h = props_t[:, 3:4]
    a = props_t[:, 4:5]

    px_ref[...] = jnp.clip(plane(0) * 0.1 * w + cx, 0.0, _IMW - 1.0)
    py_ref[...] = jnp.clip(plane(1) * 0.1 * h + cy, 0.0, _IMW - 1.0)
    pw_ref[...] = jnp.exp(jnp.minimum(plane(2) * 0.2, _CLIP)) * w
    ph_ref[...] = jnp.exp(jnp.minimum(plane(3) * 0.2, _CLIP)) * h
    pa_ref[...] = plane(4) * _R2D + a


@functools.partial(jax.jit, static_argnums=(3,))
def _run(class_logits, box_regression, proposals, block_cols):
    n = class_logits.shape[0]
    lg_t = jnp.transpose(class_logits)
    codes_t = jnp.transpose(box_regression)
    props_t = jnp.transpose(proposals)
    grid = (pl.cdiv(n, block_cols),)
    out_spec = pl.BlockSpec((block_cols, _C), lambda i: (i, 0))
    out_shape = jax.ShapeDtypeStruct((n, _C), jnp.float32)
    px, py, pw, ph, pa, scores_rm = pl.pallas_call(
        _body,
        grid=grid,
        in_specs=[
            pl.BlockSpec((_C, block_cols), lambda i: (0, i)),
            pl.BlockSpec((_C * 5, block_cols), lambda i: (0, i)),
            pl.BlockSpec((5, block_cols), lambda i: (0, i)),
        ],
        out_specs=[out_spec] * 6,
        out_shape=[out_shape] * 6,
        compiler_params=pltpu.CompilerParams(
            dimension_semantics=("parallel",),
        ),
    )(lg_t, codes_t, props_t)
    pred = jnp.stack([px, py, pw, ph, pa], axis=2)
    boxes = pred.reshape(-1, 5)
    scores = scores_rm.reshape(-1)
    return boxes, scores


def kernel(class_logits, box_regression, proposals, num_of_fwd_left=0):
    return _run(class_logits, box_regression, proposals, 2048)
